# baseline (speedup 1.0000x reference)
import functools

import jax
import jax.numpy as jnp
from jax import lax
from jax.experimental import pallas as pl
from jax.experimental.pallas import tpu as pltpu
from jax.experimental.pallas import tpu_sc as plsc

N = 10000
E = 320000
D = 128
DH = D // 2
DE = 16
NC, NS = 2, 16
NW = NC * NS
CHUNK = 64
EH = E // 2
NCHUNKS = EH // CHUNK
BASE_RPT = NCHUNKS // NW
TAIL = NCHUNKS - BASE_RPT * NW
ZROWS = (N // NS) // 8 * 8
EB = 16000
NBLK = EH // EB
BPB = EB // CHUNK
BN_EPS = 1e-5
SCALE = 2048.0
INV_SCALE = 1.0 / SCALE
QMAX = 32000.0


def _q16_pair(z):
    zi = jnp.clip(jnp.round(z * SCALE), -QMAX, QMAX).astype(jnp.int32)
    z3 = zi.reshape(BPB, CHUNK, D)
    lo = z3[:, :CHUNK // 2, :]
    hi = z3[:, CHUNK // 2:, :]
    return (lo & 0xFFFF) | (hi << 16)


def _pre_body(ea_ref, x_ref, we_ref, wx_ref, b_ref, t_ref, y_ref):
    z = (
        lax.dot_general(
            ea_ref[...], we_ref[...],
            dimension_numbers=(((0,), (0,)), ((), ())),
            preferred_element_type=jnp.float32,
        )
        + b_ref[...]
    )
    t_ref[...] = _q16_pair(z)

    @pl.when(pl.program_id(0) == 0)
    def _():
        y_ref[...] = jnp.dot(
            x_ref[...], wx_ref[...], preferred_element_type=jnp.float32
        )


def _tc_pre(half):
    def ea_map(i, _h=half):
        return (0, i + _h * NBLK)

    return pl.pallas_call(
        _pre_body,
        grid=(NBLK,),
        in_specs=[
            pl.BlockSpec((DE, EB), ea_map),
            pl.BlockSpec((N, D), lambda i: (0, 0)),
            pl.BlockSpec((DE, D), lambda i: (0, 0)),
            pl.BlockSpec((D, D), lambda i: (0, 0)),
            pl.BlockSpec((1, D), lambda i: (0, 0)),
        ],
        out_specs=[
            pl.BlockSpec((BPB, CHUNK // 2, D), lambda i: (i, 0, 0)),
            pl.BlockSpec((N, D), lambda i: (0, 0)),
        ],
        out_shape=[
            jax.ShapeDtypeStruct((NCHUNKS, CHUNK // 2, D), jnp.int32),
            jax.ShapeDtypeStruct((N, D), jnp.float32),
        ],
    )


HC = CHUNK // 2
TS = 3
DS = 6
UNROLL = 6


def _sc_body(y_hbm, t_hbm, src_hbm, dst_hbm, out_hbm,
             src_v, dst_v, t_v, y_v, agg_sh,
             ps0, ps1, ps2, pd0, pd1, pd2, pd3, pd4, pd5,
             pt0, pt1, pt2, py0, py1, py2, pc0, pc1, pc2):
    ssem = [ps0, ps1, ps2]
    dsem = [pd0, pd1, pd2, pd3, pd4, pd5]
    tsem = [pt0, pt1, pt2]
    ysem = [py0, py1, py2]
    csem = [pc0, pc1, pc2]
    c = lax.axis_index("c")
    s = lax.axis_index("s")
    w = c * NS + s
    lo = w * BASE_RPT

    def _zrow(r, carry):
        for cc in range(D // 16):
            y_v[0, r, pl.ds(cc * 16, 16)] = jnp.zeros((16,), jnp.float32)
        return carry

    lax.fori_loop(0, CHUNK, _zrow, 0)
    zbase = pl.multiple_of(s * ZROWS, 8)
    for k in range(ZROWS // CHUNK):
        pltpu.sync_copy(y_v.at[0], agg_sh.at[pl.ds(zbase + k * CHUNK, CHUNK)])
    pltpu.sync_copy(
        y_v.at[0, pl.ds(0, ZROWS % CHUNK)],
        agg_sh.at[pl.ds(zbase + (ZROWS // CHUNK) * CHUNK, ZROWS % CHUNK)],
    )

    @pl.when(s == NS - 1)
    def _():
        pltpu.sync_copy(
            y_v.at[0, pl.ds(0, N - ZROWS * NS)],
            agg_sh.at[pl.ds(ZROWS * NS, N - ZROWS * NS)],
        )

    plsc.subcore_barrier()

    def _row_slice(hbm, row):
        return hbm.at[pl.ds(pl.multiple_of(row * CHUNK, CHUNK), CHUNK)]

    def _issue_idx(j, bs, bd):
        pltpu.async_copy(_row_slice(src_hbm, lo + j), src_v.at[bs], ssem[bs])
        pltpu.async_copy(_row_slice(dst_hbm, lo + j), dst_v.at[bd], dsem[bd])

    def _wait_idx(j, bs, bd):
        pltpu.make_async_copy(_row_slice(src_hbm, lo + j), src_v.at[bs],
                              ssem[bs]).wait()
        pltpu.make_async_copy(_row_slice(dst_hbm, lo + j), dst_v.at[bd],
                              dsem[bd]).wait()

    def _issue_data(j, b):
        pltpu.async_copy(t_hbm.at[lo + j], t_v.at[b], tsem[b])
        pltpu.async_copy(y_hbm.at[src_v.at[b]], y_v.at[b], ysem[b])

    def _wait_data(j, b):
        pltpu.make_async_copy(t_hbm.at[lo + j], t_v.at[b], tsem[b]).wait()
        pltpu.make_async_copy(y_hbm.at[src_v.at[b]], y_v.at[b],
                              ysem[b]).wait()

    def _compute(b):
        def _crow(g, cr):
            for rr in range(2):
                r = g * 2 + rr
                for cc in range(D // 16):
                    sl = pl.ds(cc * 16, 16)
                    tt = t_v[b, r, sl]
                    lo_u = tt & 0xFFFF
                    lo_s = lo_u - ((lo_u & 0x8000) << 1)
                    hi_u = lax.shift_right_logical(tt, 16)
                    hi_s = hi_u - ((hi_u & 0x8000) << 1)
                    tlo = lo_s.astype(jnp.float32) * INV_SCALE
                    thi = hi_s.astype(jnp.float32) * INV_SCALE
                    y_v[b, r, sl] = jnp.maximum(y_v[b, r, sl] + tlo, 0.0)
                    y_v[b, HC + r, sl] = jnp.maximum(
                        y_v[b, HC + r, sl] + thi, 0.0
                    )
            return cr

        lax.fori_loop(0, CHUNK // 4, _crow, 0)

    def _issue_scatter(jb, db):
        pltpu.async_copy(y_v.at[jb], agg_sh.at[dst_v.at[db]], csem[jb],
                         add=True)

    def _wait_scatter(jb, db):
        pltpu.make_async_copy(y_v.at[jb], agg_sh.at[dst_v.at[db]],
                              csem[jb]).wait()

    _issue_idx(0, 0, 0)
    _issue_idx(1, 1, 1)
    _wait_idx(0, 0, 0)
    _issue_data(0, 0)

    def _steady(g, carry):
        for u in range(UNROLL):
            j = g * UNROLL + u

            @pl.when(j + 1 < BASE_RPT)
            def _():
                _wait_idx(j + 1, (u + 1) % TS, (u + 1) % DS)

            @pl.when(j >= 2)
            def _():
                _wait_scatter((u + 1) % TS, (u + 2) % DS)

            @pl.when(j + 1 < BASE_RPT)
            def _():
                _issue_data(j + 1, (u + 1) % TS)

            @pl.when(j + 2 < BASE_RPT)
            def _():
                _issue_idx(j + 2, (u + 2) % TS, (u + 2) % DS)

            _wait_data(j, u % TS)
            _compute(u % TS)
            _issue_scatter(u % TS, u % DS)
        return carry

    lax.fori_loop(0, BASE_RPT // UNROLL, _steady, 0)
    _wait_scatter((BASE_RPT - 2) % TS, (BASE_RPT - 2) % DS)
    _wait_scatter((BASE_RPT - 1) % TS, (BASE_RPT - 1) % DS)

    @pl.when(w < TAIL)
    def _():
        row = NW * BASE_RPT + w
        pltpu.sync_copy(_row_slice(src_hbm, row), src_v.at[0])
        pltpu.sync_copy(_row_slice(dst_hbm, row), dst_v.at[0])
        pltpu.sync_copy(t_hbm.at[row], t_v.at[0])
        pltpu.async_copy(y_hbm.at[src_v.at[0]], y_v.at[0], ysem[0]).wait()
        _compute(0)
        pltpu.sync_copy(y_v.at[0], agg_sh.at[dst_v.at[0]], add=True)

    plsc.subcore_barrier()

    pltpu.sync_copy(agg_sh.at[pl.ds(zbase, ZROWS)],
                    out_hbm.at[c, pl.ds(zbase, ZROWS)])

    @pl.when(s == NS - 1)
    def _():
        pltpu.sync_copy(agg_sh.at[pl.ds(ZROWS * NS, N - ZROWS * NS)],
                        out_hbm.at[c, pl.ds(ZROWS * NS, N - ZROWS * NS)])


@functools.cache
def _sc_agg():
    return functools.partial(
        pl.kernel,
        out_type=jax.ShapeDtypeStruct((NC, N, D), jnp.float32),
        mesh=plsc.VectorSubcoreMesh(
            core_axis_name="c", subcore_axis_name="s", num_cores=NC,
            num_subcores=NS,
        ),
        scratch_types=[
            pltpu.VMEM((TS, CHUNK), jnp.int32),
            pltpu.VMEM((DS, CHUNK), jnp.int32),
            pltpu.VMEM((TS, HC, D), jnp.int32),
            pltpu.VMEM((TS, CHUNK, D), jnp.float32),
            pltpu.VMEM_SHARED((N, D), jnp.float32),
        ] + [pltpu.SemaphoreType.DMA] * (TS + DS + TS + TS + TS),
    )(_sc_body)


def _post_body(x_ref, agg_a, agg_b, w1_ref, w2_ref, g_ref, be_ref, out_ref):
    h = x_ref[...] + agg_a[0] + agg_a[1] + agg_b[0] + agg_b[1]
    z = jnp.dot(h, w1_ref[...], preferred_element_type=jnp.float32)
    mu = jnp.mean(z, axis=0, keepdims=True)
    var = jnp.mean((z - mu) ** 2, axis=0, keepdims=True)
    z = (z - mu) / jnp.sqrt(var + BN_EPS)
    z = z * g_ref[...] + be_ref[...]
    z = jnp.maximum(z, 0.0)
    out_ref[...] = jnp.dot(z, w2_ref[...], preferred_element_type=jnp.float32)


def _tc_post(x, agg_a, agg_b, W1, W2, g2d, b2d):
    return pl.pallas_call(
        _post_body,
        out_shape=jax.ShapeDtypeStruct((N, D), jnp.float32),
    )(x, agg_a, agg_b, W1, W2, g2d, b2d)


def kernel(x, edge_index, edge_attr, W_edge, b_edge, W1, W2, gamma, beta):
    src = edge_index[0].astype(jnp.int32)
    dst = edge_index[1].astype(jnp.int32)
    W_x = W_edge[:D]
    W_e = W_edge[D:]
    ea_t = edge_attr.T
    b2d = b_edge.reshape(1, D)
    t_a, y = _tc_pre(0)(ea_t, x, W_e, W_x, b2d)
    t_b, _ = _tc_pre(1)(ea_t, x, W_e, W_x, b2d)
    sc = _sc_agg()
    agg_a = sc(y, t_a, src[:EH], dst[:EH])
    agg_b = sc(y, t_b, src[EH:], dst[EH:])
    return _tc_post(
        x, agg_a, agg_b, W1, W2, gamma.reshape(1, D), beta.reshape(1, D)
    )


# speedup vs baseline: 1.0009x; 1.0009x over previous
import functools

import jax
import jax.numpy as jnp
from jax import lax
from jax.experimental import pallas as pl
from jax.experimental.pallas import tpu as pltpu
from jax.experimental.pallas import tpu_sc as plsc

N = 10000
E = 320000
D = 128
DH = D // 2
DE = 16
NC, NS = 2, 16
NW = NC * NS
CHUNK = 64
EH = E // 2
NCHUNKS = EH // CHUNK
BASE_RPT = NCHUNKS // NW
TAIL = NCHUNKS - BASE_RPT * NW
ZROWS = (N // NS) // 8 * 8
EB = 16000
NBLK = EH // EB
BPB = EB // CHUNK
BN_EPS = 1e-5
SCALE = 2048.0
INV_SCALE = 1.0 / SCALE
QMAX = 32000.0


def _q16_pair(z):
    zi = jnp.clip(jnp.round(z * SCALE), -QMAX, QMAX).astype(jnp.int32)
    z3 = zi.reshape(BPB, CHUNK, D)
    lo = z3[:, :CHUNK // 2, :]
    hi = z3[:, CHUNK // 2:, :]
    return (lo & 0xFFFF) | (hi << 16)


def _pre_body(ea_ref, x_ref, we_ref, wx_ref, b_ref, t_ref, y_ref):
    z = (
        lax.dot_general(
            ea_ref[...], we_ref[...],
            dimension_numbers=(((0,), (0,)), ((), ())),
            preferred_element_type=jnp.float32,
        )
        + b_ref[...]
    )
    t_ref[...] = _q16_pair(z)

    @pl.when(pl.program_id(0) == 0)
    def _():
        y_ref[...] = jnp.dot(
            x_ref[...], wx_ref[...], preferred_element_type=jnp.float32
        )


def _tc_pre(half):
    def ea_map(i, _h=half):
        return (0, i + _h * NBLK)

    return pl.pallas_call(
        _pre_body,
        grid=(NBLK,),
        in_specs=[
            pl.BlockSpec((DE, EB), ea_map),
            pl.BlockSpec((N, D), lambda i: (0, 0)),
            pl.BlockSpec((DE, D), lambda i: (0, 0)),
            pl.BlockSpec((D, D), lambda i: (0, 0)),
            pl.BlockSpec((1, D), lambda i: (0, 0)),
        ],
        out_specs=[
            pl.BlockSpec((BPB, CHUNK // 2, D), lambda i: (i, 0, 0)),
            pl.BlockSpec((N, D), lambda i: (0, 0)),
        ],
        out_shape=[
            jax.ShapeDtypeStruct((NCHUNKS, CHUNK // 2, D), jnp.int32),
            jax.ShapeDtypeStruct((N, D), jnp.float32),
        ],
    )


HC = CHUNK // 2
TS = 3
DS = 6
UNROLL = 6


def _sc_body(y_hbm, t_hbm, src_hbm, dst_hbm, out_hbm,
             src_v, dst_v, t_v, y_v, agg_sh,
             ps0, ps1, ps2, pd0, pd1, pd2, pd3, pd4, pd5,
             pt0, pt1, pt2, py0, py1, py2, pc0, pc1, pc2):
    ssem = [ps0, ps1, ps2]
    dsem = [pd0, pd1, pd2, pd3, pd4, pd5]
    tsem = [pt0, pt1, pt2]
    ysem = [py0, py1, py2]
    csem = [pc0, pc1, pc2]
    c = lax.axis_index("c")
    s = lax.axis_index("s")
    w = c * NS + s
    lo = w * BASE_RPT

    def _zrow(r, carry):
        for cc in range(D // 16):
            y_v[0, r, pl.ds(cc * 16, 16)] = jnp.zeros((16,), jnp.float32)
        return carry

    lax.fori_loop(0, CHUNK, _zrow, 0)
    zbase = pl.multiple_of(s * ZROWS, 8)
    for k in range(ZROWS // CHUNK):
        pltpu.sync_copy(y_v.at[0], agg_sh.at[pl.ds(zbase + k * CHUNK, CHUNK)])
    pltpu.sync_copy(
        y_v.at[0, pl.ds(0, ZROWS % CHUNK)],
        agg_sh.at[pl.ds(zbase + (ZROWS // CHUNK) * CHUNK, ZROWS % CHUNK)],
    )

    @pl.when(s == NS - 1)
    def _():
        pltpu.sync_copy(
            y_v.at[0, pl.ds(0, N - ZROWS * NS)],
            agg_sh.at[pl.ds(ZROWS * NS, N - ZROWS * NS)],
        )

    plsc.subcore_barrier()

    def _row_slice(hbm, row):
        return hbm.at[pl.ds(pl.multiple_of(row * CHUNK, CHUNK), CHUNK)]

    def _issue_idx(j, bs, bd):
        pltpu.async_copy(_row_slice(src_hbm, lo + j), src_v.at[bs], ssem[bs])
        pltpu.async_copy(_row_slice(dst_hbm, lo + j), dst_v.at[bd], dsem[bd])

    def _wait_idx(j, bs, bd):
        pltpu.make_async_copy(_row_slice(src_hbm, lo + j), src_v.at[bs],
                              ssem[bs]).wait()
        pltpu.make_async_copy(_row_slice(dst_hbm, lo + j), dst_v.at[bd],
                              dsem[bd]).wait()

    def _issue_data(j, b):
        pltpu.async_copy(t_hbm.at[lo + j], t_v.at[b], tsem[b])
        pltpu.async_copy(y_hbm.at[src_v.at[b]], y_v.at[b], ysem[b])

    def _wait_data(j, b):
        pltpu.make_async_copy(t_hbm.at[lo + j], t_v.at[b], tsem[b]).wait()
        pltpu.make_async_copy(y_hbm.at[src_v.at[b]], y_v.at[b],
                              ysem[b]).wait()

    def _compute(b):
        def _crow(g, cr):
            for rr in range(2):
                r = g * 2 + rr
                for cc in range(D // 16):
                    sl = pl.ds(cc * 16, 16)
                    tt = t_v[b, r, sl]
                    lo_u = tt & 0xFFFF
                    lo_s = lo_u - ((lo_u & 0x8000) << 1)
                    hi_u = lax.shift_right_logical(tt, 16)
                    hi_s = hi_u - ((hi_u & 0x8000) << 1)
                    tlo = lo_s.astype(jnp.float32) * INV_SCALE
                    thi = hi_s.astype(jnp.float32) * INV_SCALE
                    y_v[b, r, sl] = jnp.maximum(y_v[b, r, sl] + tlo, 0.0)
                    y_v[b, HC + r, sl] = jnp.maximum(
                        y_v[b, HC + r, sl] + thi, 0.0
                    )
            return cr

        lax.fori_loop(0, CHUNK // 4, _crow, 0)

    def _issue_scatter(jb, db):
        pltpu.async_copy(y_v.at[jb], agg_sh.at[dst_v.at[db]], csem[jb],
                         add=True)

    def _wait_scatter(jb, db):
        pltpu.make_async_copy(y_v.at[jb], agg_sh.at[dst_v.at[db]],
                              csem[jb]).wait()

    _issue_idx(0, 0, 0)
    _issue_idx(1, 1, 1)
    _wait_idx(0, 0, 0)
    _issue_data(0, 0)

    def _steady(g, carry):
        for u in range(UNROLL):
            j = g * UNROLL + u

            @pl.when(j + 1 < BASE_RPT)
            def _():
                _wait_idx(j + 1, (u + 1) % TS, (u + 1) % DS)

            @pl.when(j >= 2)
            def _():
                _wait_scatter((u + 1) % TS, (u + 2) % DS)

            @pl.when(j + 1 < BASE_RPT)
            def _():
                _issue_data(j + 1, (u + 1) % TS)

            @pl.when(j + 2 < BASE_RPT)
            def _():
                _issue_idx(j + 2, (u + 2) % TS, (u + 2) % DS)

            _wait_data(j, u % TS)
            _compute(u % TS)
            _issue_scatter(u % TS, u % DS)
        return carry

    lax.fori_loop(0, BASE_RPT // UNROLL, _steady, 0)
    _wait_scatter((BASE_RPT - 2) % TS, (BASE_RPT - 2) % DS)
    _wait_scatter((BASE_RPT - 1) % TS, (BASE_RPT - 1) % DS)

    @pl.when(w < TAIL)
    def _():
        row = NW * BASE_RPT + w
        pltpu.sync_copy(_row_slice(src_hbm, row), src_v.at[0])
        pltpu.sync_copy(_row_slice(dst_hbm, row), dst_v.at[0])
        pltpu.sync_copy(t_hbm.at[row], t_v.at[0])
        pltpu.async_copy(y_hbm.at[src_v.at[0]], y_v.at[0], ysem[0]).wait()
        _compute(0)
        pltpu.sync_copy(y_v.at[0], agg_sh.at[dst_v.at[0]], add=True)

    plsc.subcore_barrier()

    pltpu.sync_copy(agg_sh.at[pl.ds(zbase, ZROWS)],
                    out_hbm.at[c, pl.ds(zbase, ZROWS)])

    @pl.when(s == NS - 1)
    def _():
        pltpu.sync_copy(agg_sh.at[pl.ds(ZROWS * NS, N - ZROWS * NS)],
                        out_hbm.at[c, pl.ds(ZROWS * NS, N - ZROWS * NS)])


@functools.cache
def _sc_agg():
    return functools.partial(
        pl.kernel,
        out_type=jax.ShapeDtypeStruct((NC, N, D), jnp.float32),
        mesh=plsc.VectorSubcoreMesh(
            core_axis_name="c", subcore_axis_name="s", num_cores=NC,
            num_subcores=NS,
        ),
        scratch_types=[
            pltpu.VMEM((TS, CHUNK), jnp.int32),
            pltpu.VMEM((DS, CHUNK), jnp.int32),
            pltpu.VMEM((TS, HC, D), jnp.int32),
            pltpu.VMEM((TS, CHUNK, D), jnp.float32),
            pltpu.VMEM_SHARED((N, D), jnp.float32),
        ] + [pltpu.SemaphoreType.DMA] * (TS + DS + TS + TS + TS),
    )(_sc_body)


def _post_body(x_ref, agg_a, agg_b, w1_ref, w2_ref, g_ref, be_ref, out_ref):
    h = x_ref[...] + agg_a[0] + agg_a[1] + agg_b[0] + agg_b[1]
    z = jnp.dot(h, w1_ref[...], preferred_element_type=jnp.float32)
    mu = jnp.mean(z, axis=0, keepdims=True)
    var = jnp.mean((z - mu) ** 2, axis=0, keepdims=True)
    z = (z - mu) / jnp.sqrt(var + BN_EPS)
    z = z * g_ref[...] + be_ref[...]
    z = jnp.maximum(z, 0.0)
    out_ref[...] = jnp.dot(z, w2_ref[...], preferred_element_type=jnp.float32)


def _tc_post(x, agg_a, agg_b, W1, W2, g2d, b2d):
    return pl.pallas_call(
        _post_body,
        out_shape=jax.ShapeDtypeStruct((N, D), jnp.float32),
    )(x, agg_a, agg_b, W1, W2, g2d, b2d)


def kernel(x, edge_index, edge_attr, W_edge, b_edge, W1, W2, gamma, beta):
    src = edge_index[0].astype(jnp.int32)
    dst = edge_index[1].astype(jnp.int32)
    W_x = W_edge[:D]
    W_e = W_edge[D:]
    ea_t = edge_attr.T
    b2d = b_edge.reshape(1, D)
    sc = _sc_agg()
    t_a, y = _tc_pre(0)(ea_t, x, W_e, W_x, b2d)
    agg_a = sc(y, t_a, src[:EH], dst[:EH])
    t_b, _ = _tc_pre(1)(ea_t, x, W_e, W_x, b2d)
    agg_b = sc(y, t_b, src[EH:], dst[EH:])
    return _tc_post(
        x, agg_a, agg_b, W1, W2, gamma.reshape(1, D), beta.reshape(1, D)
    )


# speedup vs baseline: 1.0053x; 1.0043x over previous
import functools

import jax
import jax.numpy as jnp
from jax import lax
from jax.experimental import pallas as pl
from jax.experimental.pallas import tpu as pltpu
from jax.experimental.pallas import tpu_sc as plsc

N = 10000
E = 320000
D = 128
DH = D // 2
DE = 16
NC, NS = 2, 16
NW = NC * NS
CHUNK = 64
NCHUNKS = E // CHUNK
BASE_RPT = NCHUNKS // NW
TAIL = NCHUNKS - BASE_RPT * NW
ZROWS = (N // NS) // 8 * 8
EB = 16000
BPB = EB // CHUNK
BN_EPS = 1e-5
SCALE = 2048.0
INV_SCALE = 1.0 / SCALE
QMAX = 32000.0


def _q16_pair(z):
    zi = jnp.clip(jnp.round(z * SCALE), -QMAX, QMAX).astype(jnp.int32)
    z3 = zi.reshape(BPB, CHUNK, D)
    lo = z3[:, :CHUNK // 2, :]
    hi = z3[:, CHUNK // 2:, :]
    return (lo & 0xFFFF) | (hi << 16)


def _pre_body(ea_ref, x_ref, we_ref, wx_ref, b_ref, t_ref, y_ref):
    z = (
        lax.dot_general(
            ea_ref[...], we_ref[...],
            dimension_numbers=(((0,), (0,)), ((), ())),
            preferred_element_type=jnp.float32,
        )
        + b_ref[...]
    )
    t_ref[...] = _q16_pair(z)

    @pl.when(pl.program_id(0) == 0)
    def _():
        y_ref[...] = jnp.dot(
            x_ref[...], wx_ref[...], preferred_element_type=jnp.float32
        )


def _tc_pre(edge_attr_t, x, W_e, W_x, b2d):
    return pl.pallas_call(
        _pre_body,
        grid=(E // EB,),
        in_specs=[
            pl.BlockSpec((DE, EB), lambda i: (0, i)),
            pl.BlockSpec((N, D), lambda i: (0, 0)),
            pl.BlockSpec((DE, D), lambda i: (0, 0)),
            pl.BlockSpec((D, D), lambda i: (0, 0)),
            pl.BlockSpec((1, D), lambda i: (0, 0)),
        ],
        out_specs=[
            pl.BlockSpec((BPB, CHUNK // 2, D), lambda i: (i, 0, 0)),
            pl.BlockSpec((N, D), lambda i: (0, 0)),
        ],
        out_shape=[
            jax.ShapeDtypeStruct((NCHUNKS, CHUNK // 2, D), jnp.int32),
            jax.ShapeDtypeStruct((N, D), jnp.float32),
        ],
    )(edge_attr_t, x, W_e, W_x, b2d)


HC = CHUNK // 2
TS = 3
DS = 4
UNROLL = TS * DS


def _sc_body(y_hbm, t_hbm, src_hbm, dst_hbm, out_hbm,
             src_v, dst_v, t_v, y_v, agg_sh,
             ps0, ps1, ps2, pd0, pd1, pd2, pd3,
             pt0, pt1, pt2, py0, py1, py2, pc0, pc1, pc2):
    ssem = [ps0, ps1, ps2]
    dsem = [pd0, pd1, pd2, pd3]
    tsem = [pt0, pt1, pt2]
    ysem = [py0, py1, py2]
    csem = [pc0, pc1, pc2]
    c = lax.axis_index("c")
    s = lax.axis_index("s")
    w = c * NS + s
    lo = w * BASE_RPT

    def _zrow(r, carry):
        for cc in range(D // 16):
            y_v[0, r, pl.ds(cc * 16, 16)] = jnp.zeros((16,), jnp.float32)
        return carry

    lax.fori_loop(0, CHUNK, _zrow, 0)
    zbase = pl.multiple_of(s * ZROWS, 8)
    for k in range(ZROWS // CHUNK):
        pltpu.sync_copy(y_v.at[0], agg_sh.at[pl.ds(zbase + k * CHUNK, CHUNK)])
    pltpu.sync_copy(
        y_v.at[0, pl.ds(0, ZROWS % CHUNK)],
        agg_sh.at[pl.ds(zbase + (ZROWS // CHUNK) * CHUNK, ZROWS % CHUNK)],
    )

    @pl.when(s == NS - 1)
    def _():
        pltpu.sync_copy(
            y_v.at[0, pl.ds(0, N - ZROWS * NS)],
            agg_sh.at[pl.ds(ZROWS * NS, N - ZROWS * NS)],
        )

    plsc.subcore_barrier()

    def _row_slice(hbm, row):
        return hbm.at[pl.ds(pl.multiple_of(row * CHUNK, CHUNK), CHUNK)]

    def _issue_idx(j, bs, bd):
        pltpu.async_copy(_row_slice(src_hbm, lo + j), src_v.at[bs], ssem[bs])
        pltpu.async_copy(_row_slice(dst_hbm, lo + j), dst_v.at[bd], dsem[bd])

    def _wait_idx(j, bs, bd):
        pltpu.make_async_copy(_row_slice(src_hbm, lo + j), src_v.at[bs],
                              ssem[bs]).wait()
        pltpu.make_async_copy(_row_slice(dst_hbm, lo + j), dst_v.at[bd],
                              dsem[bd]).wait()

    def _issue_data(j, b):
        pltpu.async_copy(t_hbm.at[lo + j], t_v.at[b], tsem[b])
        pltpu.async_copy(y_hbm.at[src_v.at[b]], y_v.at[b], ysem[b])

    def _wait_data(j, b):
        pltpu.make_async_copy(t_hbm.at[lo + j], t_v.at[b], tsem[b]).wait()
        pltpu.make_async_copy(y_hbm.at[src_v.at[b]], y_v.at[b],
                              ysem[b]).wait()

    def _compute(b):
        def _crow(g, cr):
            for rr in range(2):
                r = g * 2 + rr
                for cc in range(D // 16):
                    sl = pl.ds(cc * 16, 16)
                    tt = t_v[b, r, sl]
                    lo_u = tt & 0xFFFF
                    lo_s = lo_u - ((lo_u & 0x8000) << 1)
                    hi_u = lax.shift_right_logical(tt, 16)
                    hi_s = hi_u - ((hi_u & 0x8000) << 1)
                    tlo = lo_s.astype(jnp.float32) * INV_SCALE
                    thi = hi_s.astype(jnp.float32) * INV_SCALE
                    y_v[b, r, sl] = jnp.maximum(y_v[b, r, sl] + tlo, 0.0)
                    y_v[b, HC + r, sl] = jnp.maximum(
                        y_v[b, HC + r, sl] + thi, 0.0
                    )
            return cr

        lax.fori_loop(0, CHUNK // 4, _crow, 0)

    def _issue_scatter(jb, db):
        pltpu.async_copy(y_v.at[jb], agg_sh.at[dst_v.at[db]], csem[jb],
                         add=True)

    def _wait_scatter(jb, db):
        pltpu.make_async_copy(y_v.at[jb], agg_sh.at[dst_v.at[db]],
                              csem[jb]).wait()

    _issue_idx(0, 0, 0)
    _issue_idx(1, 1, 1)
    _wait_idx(0, 0, 0)
    _issue_data(0, 0)

    def _steady(g, carry):
        for u in range(UNROLL):
            j = g * UNROLL + u

            @pl.when(j + 1 < BASE_RPT)
            def _():
                _wait_idx(j + 1, (u + 1) % TS, (u + 1) % DS)

            @pl.when(j >= 2)
            def _():
                _wait_scatter((u + 1) % TS, (u + 2) % DS)

            @pl.when(j + 1 < BASE_RPT)
            def _():
                _issue_data(j + 1, (u + 1) % TS)

            @pl.when(j + 2 < BASE_RPT)
            def _():
                _issue_idx(j + 2, (u + 2) % TS, (u + 2) % DS)

            _wait_data(j, u % TS)
            _compute(u % TS)
            _issue_scatter(u % TS, u % DS)
        return carry

    lax.fori_loop(0, BASE_RPT // UNROLL, _steady, 0)
    _wait_scatter((BASE_RPT - 2) % TS, (BASE_RPT - 2) % DS)
    _wait_scatter((BASE_RPT - 1) % TS, (BASE_RPT - 1) % DS)

    @pl.when(w < TAIL)
    def _():
        row = NW * BASE_RPT + w
        pltpu.sync_copy(_row_slice(src_hbm, row), src_v.at[0])
        pltpu.sync_copy(_row_slice(dst_hbm, row), dst_v.at[0])
        pltpu.sync_copy(t_hbm.at[row], t_v.at[0])
        pltpu.async_copy(y_hbm.at[src_v.at[0]], y_v.at[0], ysem[0]).wait()
        _compute(0)
        pltpu.sync_copy(y_v.at[0], agg_sh.at[dst_v.at[0]], add=True)

    plsc.subcore_barrier()

    pltpu.sync_copy(agg_sh.at[pl.ds(zbase, ZROWS)],
                    out_hbm.at[c, pl.ds(zbase, ZROWS)])

    @pl.when(s == NS - 1)
    def _():
        pltpu.sync_copy(agg_sh.at[pl.ds(ZROWS * NS, N - ZROWS * NS)],
                        out_hbm.at[c, pl.ds(ZROWS * NS, N - ZROWS * NS)])


@functools.cache
def _sc_agg():
    return functools.partial(
        pl.kernel,
        out_type=jax.ShapeDtypeStruct((NC, N, D), jnp.float32),
        mesh=plsc.VectorSubcoreMesh(
            core_axis_name="c", subcore_axis_name="s", num_cores=NC,
            num_subcores=NS,
        ),
        scratch_types=[
            pltpu.VMEM((TS, CHUNK), jnp.int32),
            pltpu.VMEM((DS, CHUNK), jnp.int32),
            pltpu.VMEM((TS, HC, D), jnp.int32),
            pltpu.VMEM((TS, CHUNK, D), jnp.float32),
            pltpu.VMEM_SHARED((N, D), jnp.float32),
        ] + [pltpu.SemaphoreType.DMA] * (TS + DS + TS + TS + TS),
    )(_sc_body)


def _post_body(x_ref, agg_ref, w1_ref, w2_ref, g_ref, be_ref, out_ref):
    h = x_ref[...] + agg_ref[0] + agg_ref[1]
    z = jnp.dot(h, w1_ref[...], preferred_element_type=jnp.float32)
    mu = jnp.mean(z, axis=0, keepdims=True)
    var = jnp.mean((z - mu) ** 2, axis=0, keepdims=True)
    z = (z - mu) / jnp.sqrt(var + BN_EPS)
    z = z * g_ref[...] + be_ref[...]
    z = jnp.maximum(z, 0.0)
    out_ref[...] = jnp.dot(z, w2_ref[...], preferred_element_type=jnp.float32)


def _tc_post(x, agg, W1, W2, g2d, b2d):
    return pl.pallas_call(
        _post_body,
        out_shape=jax.ShapeDtypeStruct((N, D), jnp.float32),
    )(x, agg, W1, W2, g2d, b2d)


def kernel(x, edge_index, edge_attr, W_edge, b_edge, W1, W2, gamma, beta):
    src = edge_index[0].astype(jnp.int32)
    dst = edge_index[1].astype(jnp.int32)
    W_x = W_edge[:D]
    W_e = W_edge[D:]
    t, y = _tc_pre(edge_attr.T, x, W_e, W_x, b_edge.reshape(1, D))
    agg = _sc_agg()(y, t, src, dst)
    return _tc_post(
        x, agg, W1, W2, gamma.reshape(1, D), beta.reshape(1, D)
    )
